# baseline (speedup 1.0000x reference)
import functools

import jax
import jax.numpy as jnp
from jax import lax
from jax.experimental import pallas as pl
from jax.experimental.pallas import tpu as pltpu
from jax.experimental.pallas import tpu_sc as plsc

_TOKENS = 8192
_DMODEL = 4096
_EXPERTS = 64
_K = 8
_BLOCK = 512

_NEG_INF = float("-inf")

_NC = 2
_NS = 16
_L = 16
_NW = _NC * _NS
_RPW = _TOKENS // _NW
_NT = _RPW // _L


def _logits_block(x_ref, w_ref, b_ref, out_ref):
    out_ref[...] = lax.dot_general(
        x_ref[...], w_ref[...],
        dimension_numbers=(((1,), (1,)), ((), ())),
        preferred_element_type=jnp.float32,
    ) + b_ref[...][None, :]


def _compute_logits(x, W, b):
    grid = _TOKENS // _BLOCK
    return pl.pallas_call(
        _logits_block,
        grid=(grid,),
        in_specs=[
            pl.BlockSpec((_BLOCK, _DMODEL), lambda i: (i, 0)),
            pl.BlockSpec((_EXPERTS, _DMODEL), lambda i: (0, 0)),
            pl.BlockSpec((_EXPERTS,), lambda i: (0,)),
        ],
        out_specs=pl.BlockSpec((_BLOCK, _EXPERTS), lambda i: (i, 0)),
        out_shape=jax.ShapeDtypeStruct((_TOKENS, _EXPERTS), jnp.float32),
        compiler_params=pltpu.CompilerParams(
            dimension_semantics=("arbitrary",),
        ),
    )(x, W, b)


def _router_sc(logits_hbm, gates_hbm, idx_hbm, lbuf, gbuf, ibuf):
    wid = lax.axis_index("s") * _NC + lax.axis_index("c")
    base = wid * _RPW
    pltpu.sync_copy(logits_hbm.at[pl.ds(base * _EXPERTS, _RPW * _EXPERTS)],
                    lbuf)

    iota = lax.broadcasted_iota(jnp.int32, (_L,), 0)
    neg_inf = jnp.full((_L,), _NEG_INF, jnp.float32)
    zero = jnp.zeros((_L,), jnp.float32)

    def tile_body(t, carry):
        raddr = (t * _L + iota) * _EXPERTS
        raddr_k = (t * _L + iota) * _K
        vals = []
        idxs = []
        for _ in range(_K):
            m = neg_inf
            a = jnp.zeros((_L,), jnp.int32)
            for e in range(_EXPERTS):
                c = plsc.load_gather(lbuf, [raddr + e])
                gt = c > m
                m = jnp.where(gt, c, m)
                a = jnp.where(gt, e, a)
            vals.append(m)
            idxs.append(a)
            plsc.store_scatter(lbuf, [raddr + a], neg_inf)

        exps = [jnp.exp(v - vals[0]) for v in vals]
        denom = exps[0]
        for v in exps[1:]:
            denom = denom + v

        for e in range(_EXPERTS):
            plsc.store_scatter(gbuf, [raddr + e], zero)
        for k in range(_K):
            plsc.store_scatter(gbuf, [raddr + idxs[k]], exps[k] / denom)
            plsc.store_scatter(ibuf, [raddr_k + k], idxs[k])
        return carry

    lax.fori_loop(0, _NT, tile_body, 0)
    pltpu.sync_copy(gbuf, gates_hbm.at[pl.ds(base * _EXPERTS,
                                             _RPW * _EXPERTS)])
    pltpu.sync_copy(ibuf, idx_hbm.at[pl.ds(base * _K, _RPW * _K)])


def _route(logits_flat):
    f = functools.partial(
        pl.kernel,
        out_type=[
            jax.ShapeDtypeStruct((_TOKENS * _EXPERTS,), jnp.float32),
            jax.ShapeDtypeStruct((_TOKENS * _K,), jnp.int32),
        ],
        mesh=plsc.VectorSubcoreMesh(core_axis_name="c", subcore_axis_name="s"),
        compiler_params=pltpu.CompilerParams(needs_layout_passes=False),
        scratch_types=[
            pltpu.VMEM((_RPW * _EXPERTS,), jnp.float32),
            pltpu.VMEM((_RPW * _EXPERTS,), jnp.float32),
            pltpu.VMEM((_RPW * _K,), jnp.int32),
        ],
    )(_router_sc)
    return f(logits_flat)


def kernel(x, W, b):
    logits = _compute_logits(x, W, b)
    gates_flat, idx_flat = _route(logits.reshape(_TOKENS * _EXPERTS))
    return (gates_flat.reshape(_TOKENS, _EXPERTS),
            idx_flat.reshape(_TOKENS, _K))


# speedup vs baseline: 1.6042x; 1.6042x over previous
import functools

import jax
import jax.numpy as jnp
from jax import lax
from jax.experimental import pallas as pl
from jax.experimental.pallas import tpu as pltpu
from jax.experimental.pallas import tpu_sc as plsc

_TOKENS = 8192
_DMODEL = 4096
_EXPERTS = 64
_K = 8
_BLOCK = 512

_NEG_INF = float("-inf")

_NC = 2
_NS = 16
_L = 16
_NW = _NC * _NS
_RPW = _TOKENS // _NW
_NT = _RPW // _L


def _logits_block(x_ref, w_ref, b_ref, out_ref):
    out_ref[...] = lax.dot_general(
        x_ref[...], w_ref[...],
        dimension_numbers=(((1,), (1,)), ((), ())),
        preferred_element_type=jnp.float32,
    ) + b_ref[...][None, :]


def _compute_logits(x, W, b):
    grid = _TOKENS // _BLOCK
    return pl.pallas_call(
        _logits_block,
        grid=(grid,),
        in_specs=[
            pl.BlockSpec((_BLOCK, _DMODEL), lambda i: (i, 0)),
            pl.BlockSpec((_EXPERTS, _DMODEL), lambda i: (0, 0)),
            pl.BlockSpec((_EXPERTS,), lambda i: (0,)),
        ],
        out_specs=pl.BlockSpec((_BLOCK, _EXPERTS), lambda i: (i, 0)),
        out_shape=jax.ShapeDtypeStruct((_TOKENS, _EXPERTS), jnp.float32),
        compiler_params=pltpu.CompilerParams(
            dimension_semantics=("arbitrary",),
        ),
    )(x, W, b)


_NG = 8
_GS = _EXPERTS // _NG


def _tree_max(vals, idxs):
    while len(vals) > 1:
        nv, ni = [], []
        for p in range(0, len(vals), 2):
            gt = vals[p + 1] > vals[p]
            nv.append(jnp.where(gt, vals[p + 1], vals[p]))
            ni.append(jnp.where(gt, idxs[p + 1], idxs[p]))
        vals, idxs = nv, ni
    return vals[0], idxs[0]


def _router_sc(logits_hbm, gates_hbm, idx_hbm, lbuf, gbuf, ibuf):
    wid = lax.axis_index("s") * _NC + lax.axis_index("c")
    base = wid * _RPW
    pltpu.sync_copy(logits_hbm.at[pl.ds(base * _EXPERTS, _RPW * _EXPERTS)],
                    lbuf)

    iota = lax.broadcasted_iota(jnp.int32, (_L,), 0)
    neg_inf = jnp.full((_L,), _NEG_INF, jnp.float32)
    zero = jnp.zeros((_L,), jnp.float32)

    for off in range(0, _RPW * _EXPERTS, _L):
        gbuf[pl.ds(off, _L)] = zero

    def tile_body(t, carry):
        raddr = (t * _L + iota) * _EXPERTS
        raddr_k = (t * _L + iota) * _K

        gm, ga = [], []
        for j in range(_NG):
            cols = [plsc.load_gather(lbuf, [raddr + (j * _GS + i)])
                    for i in range(_GS)]
            cidx = [jnp.full((_L,), j * _GS + i, jnp.int32)
                    for i in range(_GS)]
            v, a = _tree_max(cols, cidx)
            gm.append(v)
            ga.append(a)

        vals, idxs = [], []
        for k in range(_K):
            m_k, wg = _tree_max(list(gm),
                                [jnp.full((_L,), j, jnp.int32)
                                 for j in range(_NG)])
            a_k = ga[0]
            for j in range(1, _NG):
                a_k = jnp.where(wg == j, ga[j], a_k)
            vals.append(m_k)
            idxs.append(a_k)
            if k == _K - 1:
                break
            plsc.store_scatter(lbuf, [raddr + a_k], neg_inf)
            gbase = raddr + wg * _GS
            cols = [plsc.load_gather(lbuf, [gbase + i]) for i in range(_GS)]
            cidx = [wg * _GS + i for i in range(_GS)]
            v, a = _tree_max(cols, cidx)
            for j in range(_NG):
                p = wg == j
                gm[j] = jnp.where(p, v, gm[j])
                ga[j] = jnp.where(p, a, ga[j])

        exps = [jnp.exp(v - vals[0]) for v in vals]
        denom = (exps[0] + exps[1]) + (exps[2] + exps[3])
        denom = denom + ((exps[4] + exps[5]) + (exps[6] + exps[7]))

        for k in range(_K):
            plsc.store_scatter(gbuf, [raddr + idxs[k]], exps[k] / denom)
            plsc.store_scatter(ibuf, [raddr_k + k], idxs[k])
        return carry

    lax.fori_loop(0, _NT, tile_body, 0)
    pltpu.sync_copy(gbuf, gates_hbm.at[pl.ds(base * _EXPERTS,
                                             _RPW * _EXPERTS)])
    pltpu.sync_copy(ibuf, idx_hbm.at[pl.ds(base * _K, _RPW * _K)])


def _route(logits_flat):
    f = functools.partial(
        pl.kernel,
        out_type=[
            jax.ShapeDtypeStruct((_TOKENS * _EXPERTS,), jnp.float32),
            jax.ShapeDtypeStruct((_TOKENS * _K,), jnp.int32),
        ],
        mesh=plsc.VectorSubcoreMesh(core_axis_name="c", subcore_axis_name="s"),
        compiler_params=pltpu.CompilerParams(needs_layout_passes=False),
        scratch_types=[
            pltpu.VMEM((_RPW * _EXPERTS,), jnp.float32),
            pltpu.VMEM((_RPW * _EXPERTS,), jnp.float32),
            pltpu.VMEM((_RPW * _K,), jnp.int32),
        ],
    )(_router_sc)
    return f(logits_flat)


def kernel(x, W, b):
    logits = _compute_logits(x, W, b)
    gates_flat, idx_flat = _route(logits.reshape(_TOKENS * _EXPERTS))
    return (gates_flat.reshape(_TOKENS, _EXPERTS),
            idx_flat.reshape(_TOKENS, _K))
